# Initial kernel scaffold; baseline (speedup 1.0000x reference)
import jax
import jax.numpy as jnp
from jax.experimental import pallas as pl


def kernel(node_feat, edge_index, W, b):
    raise NotImplementedError("write your pallas kernel here")


# speedup vs baseline: 14.4035x; 14.4035x over previous
import functools

import jax
import jax.numpy as jnp
from jax import lax
from jax.experimental import pallas as pl
from jax.experimental.pallas import tpu as pltpu
from jax.experimental.pallas import tpu_sc as plsc

NC = 2
NS = 16
L = 16
NW = NC * NS

K = 80


def _sc_segment_sum(x, eidx, NP):
  N, D = x.shape
  nw, nchunk, two, k = eidx.shape
  assert nw == NW and k == K and two == 2
  rpt = NP // NS

  mesh = plsc.VectorSubcoreMesh(core_axis_name="c", subcore_axis_name="s")

  @functools.partial(
      pl.kernel,
      out_type=[
          jax.ShapeDtypeStruct((NC, NP, D), jnp.float32),
          jax.ShapeDtypeStruct((NC * NP,), jnp.float32),
      ],
      mesh=mesh,
      scratch_types=[
          pltpu.VMEM_SHARED((NP, D), jnp.float32),
          pltpu.VMEM_SHARED((NP,), jnp.float32),
          pltpu.VMEM((4, 2, K), jnp.int32),
          pltpu.VMEM((K, D), jnp.float32),
          pltpu.VMEM((K, D), jnp.float32),
          pltpu.VMEM((K,), jnp.float32),
          pltpu.VMEM((rpt,), jnp.float32),
          pltpu.SemaphoreType.DMA,
          pltpu.SemaphoreType.DMA,
          pltpu.SemaphoreType.DMA,
          pltpu.SemaphoreType.DMA,
          pltpu.SemaphoreType.DMA,
          pltpu.SemaphoreType.DMA,
      ],
  )
  def seg_sum(x_hbm, eidx_hbm, s_out, deg_out, acc, dacc, ering, rows0, rows1,
              ones_v, zflat, gsem0, gsem1, isem0, isem1, isem2, isem3):
    cid = lax.axis_index("c")
    sid = lax.axis_index("s")
    wid = sid * NC + cid
    gbufs = (rows0, rows1)
    gsems = (gsem0, gsem1)
    isems = (isem0, isem1, isem2, isem3)

    zv = jnp.zeros((L,), jnp.float32)
    ov = jnp.ones((L,), jnp.float32)

    @pl.loop(0, K)
    def _(r):
      for j in range(D // L):
        rows0[r, pl.ds(j * L, L)] = zv

    for j in range(K // L):
      ones_v[pl.ds(j * L, L)] = ov
    for j in range(rpt // L):
      zflat[pl.ds(j * L, L)] = zv

    for j in range(rpt // K):
      pltpu.sync_copy(rows0, acc.at[pl.ds(sid * rpt + j * K, K)])
    pltpu.sync_copy(zflat, dacc.at[pl.ds(sid * rpt, rpt)])
    plsc.subcore_barrier()

    def start_idx(c, s):
      pltpu.async_copy(eidx_hbm.at[wid, c], ering.at[s], isems[s])

    def start_gather(s, b):
      pltpu.make_async_copy(eidx_hbm.at[wid, 0], ering.at[s], isems[s]).wait()
      pltpu.async_copy(x_hbm.at[ering.at[s, 0]], gbufs[b], gsems[b])

    def accumulate(s, b):
      pltpu.make_async_copy(x_hbm.at[ering.at[s, 0]], gbufs[b], gsems[b]).wait()
      pltpu.sync_copy(gbufs[b], acc.at[ering.at[s, 1]], add=True)
      pltpu.sync_copy(ones_v, dacc.at[ering.at[s, 1]], add=True)

    for c in range(4):
      start_idx(c, c)
    for c in range(2):
      start_gather(c, c)

    main = nchunk - 5
    assert main % 4 == 0

    @pl.loop(0, main, step=4)
    def _(i):
      for b in range(4):
        accumulate(b, b % 2)
        start_idx(i + b + 4, b)
        start_gather((b + 2) % 4, b % 2)

    accumulate(0, 0)
    start_idx(main + 4, 0)
    start_gather(2, 0)
    accumulate(1, 1)
    start_gather(3, 1)
    accumulate(2, 0)
    start_gather(0, 0)
    accumulate(3, 1)
    accumulate(0, 0)

    plsc.subcore_barrier()

    sl = pl.ds(sid * rpt, rpt)
    pltpu.sync_copy(acc.at[sl], s_out.at[cid, sl])
    pltpu.sync_copy(dacc.at[sl], deg_out.at[pl.ds(cid * NP + sid * rpt, rpt)])

  return seg_sum(x, eidx)


def _dense(x, sp, degp, W, b2, R):
  N, D = x.shape
  grid = N // R

  def body(x_ref, sp_ref, degp_ref, w_ref, b_ref, o_ref):
    xb = x_ref[...]
    s = sp_ref[0] + sp_ref[1]
    deg = degp_ref[0, 0] + degp_ref[1, 0]
    w1 = w_ref[0:D]
    w2 = w_ref[D:2 * D]
    t = jnp.dot(s, w1, preferred_element_type=jnp.float32,
                precision=lax.Precision.HIGHEST)
    u = jnp.dot(xb, w2, preferred_element_type=jnp.float32,
                precision=lax.Precision.HIGHEST) + b_ref[0]
    z = t + deg * u
    sig = 1.0 / (1.0 + jnp.exp(-z))
    softplus = jnp.maximum(xb, 0.0) + jnp.log1p(jnp.exp(-jnp.abs(xb)))
    o_ref[...] = sig + softplus

  return pl.pallas_call(
      body,
      grid=(grid,),
      in_specs=[
          pl.BlockSpec((R, D), lambda i: (i, 0)),
          pl.BlockSpec((NC, R, D), lambda i: (0, i, 0)),
          pl.BlockSpec((NC, 1, R, 1), lambda i: (0, i, 0, 0)),
          pl.BlockSpec((2 * D, D), lambda i: (0, 0)),
          pl.BlockSpec((1, D), lambda i: (0, 0)),
      ],
      out_specs=pl.BlockSpec((R, D), lambda i: (i, 0)),
      out_shape=jax.ShapeDtypeStruct((N, D), jnp.float32),
  )(x, sp, degp, W, b2)


def kernel(node_feat, edge_index, W, b):
  N, D = node_feat.shape
  E = edge_index.shape[1]
  epw = E // NW
  nchunk = epw // K
  NP = -(-N // (NS * K)) * (NS * K)
  R = 1000

  eidx = jnp.stack(
      [edge_index[0].reshape(NW, nchunk, K),
       edge_index[1].reshape(NW, nchunk, K)], axis=2)

  sp, degp = _sc_segment_sum(node_feat, eidx, NP)
  degp4 = degp.reshape(NC, NP)[:, :N].reshape(NC, N // R, R, 1)
  return _dense(node_feat, sp, degp4, W, b.reshape(1, D), R)


# speedup vs baseline: 14.9937x; 1.0410x over previous
import functools

import jax
import jax.numpy as jnp
from jax import lax
from jax.experimental import pallas as pl
from jax.experimental.pallas import tpu as pltpu
from jax.experimental.pallas import tpu_sc as plsc

NC = 2
NS = 16
L = 16
NW = NC * NS

K = 80


def _sc_segment_sum(x, eidx, NP):
  N, D = x.shape
  nw, nchunk, two, k = eidx.shape
  assert nw == NW and k == K and two == 2
  rpt = NP // NS

  mesh = plsc.VectorSubcoreMesh(core_axis_name="c", subcore_axis_name="s")

  @functools.partial(
      pl.kernel,
      out_type=[
          jax.ShapeDtypeStruct((NC, NP, D), jnp.float32),
          jax.ShapeDtypeStruct((NC * NP,), jnp.float32),
      ],
      mesh=mesh,
      scratch_types=[
          pltpu.VMEM_SHARED((NP, D), jnp.float32),
          pltpu.VMEM_SHARED((NP,), jnp.float32),
          pltpu.VMEM((4, 2, K), jnp.int32),
          pltpu.VMEM((K, D), jnp.float32),
          pltpu.VMEM((K, D), jnp.float32),
          pltpu.VMEM((K,), jnp.float32),
          pltpu.VMEM((rpt,), jnp.float32),
          pltpu.SemaphoreType.DMA,
          pltpu.SemaphoreType.DMA,
          pltpu.SemaphoreType.DMA,
          pltpu.SemaphoreType.DMA,
          pltpu.SemaphoreType.DMA,
          pltpu.SemaphoreType.DMA,
      ],
  )
  def seg_sum(x_hbm, eidx_hbm, s_out, deg_out, acc, dacc, ering, rows0, rows1,
              ones_v, zflat, gsem0, gsem1, isem0, isem1, isem2, isem3):
    cid = lax.axis_index("c")
    sid = lax.axis_index("s")
    wid = sid * NC + cid
    gbufs = (rows0, rows1)
    gsems = (gsem0, gsem1)
    isems = (isem0, isem1, isem2, isem3)

    zv = jnp.zeros((L,), jnp.float32)
    ov = jnp.ones((L,), jnp.float32)

    @pl.loop(0, K)
    def _(r):
      for j in range(D // L):
        rows0[r, pl.ds(j * L, L)] = zv

    for j in range(K // L):
      ones_v[pl.ds(j * L, L)] = ov
    for j in range(rpt // L):
      zflat[pl.ds(j * L, L)] = zv

    for j in range(rpt // K):
      pltpu.sync_copy(rows0, acc.at[pl.ds(sid * rpt + j * K, K)])
    pltpu.sync_copy(zflat, dacc.at[pl.ds(sid * rpt, rpt)])
    plsc.subcore_barrier()

    def start_idx(c, s):
      pltpu.async_copy(eidx_hbm.at[wid, c], ering.at[s], isems[s])

    def start_gather(s, b):
      pltpu.make_async_copy(eidx_hbm.at[wid, 0], ering.at[s], isems[s]).wait()
      pltpu.async_copy(x_hbm.at[ering.at[s, 0]], gbufs[b], gsems[b])

    def accumulate(s, b):
      pltpu.make_async_copy(x_hbm.at[ering.at[s, 0]], gbufs[b], gsems[b]).wait()
      pltpu.sync_copy(gbufs[b], acc.at[ering.at[s, 1]], add=True)
      pltpu.sync_copy(ones_v, dacc.at[ering.at[s, 1]], add=True)

    for c in range(4):
      start_idx(c, c)
    for c in range(2):
      start_gather(c, c)

    main = nchunk - 5
    assert main % 4 == 0

    @pl.loop(0, main, step=4)
    def _(i):
      for b in range(4):
        accumulate(b, b % 2)
        start_idx(i + b + 4, b)
        start_gather((b + 2) % 4, b % 2)

    accumulate(0, 0)
    start_idx(main + 4, 0)
    start_gather(2, 0)
    accumulate(1, 1)
    start_gather(3, 1)
    accumulate(2, 0)
    start_gather(0, 0)
    accumulate(3, 1)
    accumulate(0, 0)

    plsc.subcore_barrier()

    sl = pl.ds(sid * rpt, rpt)
    pltpu.sync_copy(acc.at[sl], s_out.at[cid, sl])
    pltpu.sync_copy(dacc.at[sl], deg_out.at[pl.ds(cid * NP + sid * rpt, rpt)])

  return seg_sum(x, eidx)


def _dense(x, sp, degp, W, b2, R):
  N, D = x.shape
  grid = N // R

  def body(x_ref, sp_ref, degp_ref, w_ref, b_ref, o_ref):
    xb = x_ref[...]
    s = sp_ref[0] + sp_ref[1]
    deg = degp_ref[0, 0] + degp_ref[1, 0]
    w1 = w_ref[0:D]
    w2 = w_ref[D:2 * D]
    t = jnp.dot(s, w1, preferred_element_type=jnp.float32)
    u = jnp.dot(xb, w2, preferred_element_type=jnp.float32) + b_ref[0]
    z = t + deg * u
    sig = 1.0 / (1.0 + jnp.exp(-z))
    softplus = jnp.maximum(xb, 0.0) + jnp.log1p(jnp.exp(-jnp.abs(xb)))
    o_ref[...] = sig + softplus

  return pl.pallas_call(
      body,
      grid=(grid,),
      in_specs=[
          pl.BlockSpec((R, D), lambda i: (i, 0)),
          pl.BlockSpec((NC, R, D), lambda i: (0, i, 0)),
          pl.BlockSpec((NC, 1, R, 1), lambda i: (0, i, 0, 0)),
          pl.BlockSpec((2 * D, D), lambda i: (0, 0)),
          pl.BlockSpec((1, D), lambda i: (0, 0)),
      ],
      out_specs=pl.BlockSpec((R, D), lambda i: (i, 0)),
      out_shape=jax.ShapeDtypeStruct((N, D), jnp.float32),
  )(x, sp, degp, W, b2)


def kernel(node_feat, edge_index, W, b):
  N, D = node_feat.shape
  E = edge_index.shape[1]
  epw = E // NW
  nchunk = epw // K
  NP = -(-N // (NS * K)) * (NS * K)
  R = 1000

  eidx = jnp.stack(
      [edge_index[0].reshape(NW, nchunk, K),
       edge_index[1].reshape(NW, nchunk, K)], axis=2)

  sp, degp = _sc_segment_sum(node_feat, eidx, NP)
  degp4 = degp.reshape(NC, NP)[:, :N].reshape(NC, N // R, R, 1)
  return _dense(node_feat, sp, degp4, W, b.reshape(1, D), R)


# speedup vs baseline: 16.2795x; 1.0858x over previous
import functools

import jax
import jax.numpy as jnp
from jax import lax
from jax.experimental import pallas as pl
from jax.experimental.pallas import tpu as pltpu
from jax.experimental.pallas import tpu_sc as plsc

NC = 2
NS = 16
L = 16
NW = NC * NS

K = 80


def _sc_segment_sum(x, src1d, dst1d, NP):
  N, D = x.shape
  nchunk = src1d.shape[0] // (NW * K)
  rpt = NP // NS

  mesh = plsc.VectorSubcoreMesh(core_axis_name="c", subcore_axis_name="s")

  @functools.partial(
      pl.kernel,
      out_type=[
          jax.ShapeDtypeStruct((NC, NP, D), jnp.float32),
          jax.ShapeDtypeStruct((NC * NP,), jnp.float32),
      ],
      mesh=mesh,
      scratch_types=[
          pltpu.VMEM_SHARED((NP, D), jnp.float32),
          pltpu.VMEM_SHARED((NP,), jnp.float32),
          pltpu.VMEM((4, K), jnp.int32),
          pltpu.VMEM((4, K), jnp.int32),
          pltpu.VMEM((K, D), jnp.float32),
          pltpu.VMEM((K, D), jnp.float32),
          pltpu.VMEM((K,), jnp.float32),
          pltpu.VMEM((rpt,), jnp.float32),
          pltpu.SemaphoreType.DMA,
          pltpu.SemaphoreType.DMA,
          pltpu.SemaphoreType.DMA,
          pltpu.SemaphoreType.DMA,
          pltpu.SemaphoreType.DMA,
          pltpu.SemaphoreType.DMA,
      ],
  )
  def seg_sum(x_hbm, src_hbm, dst_hbm, s_out, deg_out, acc, dacc,
              sring, dring, rows0, rows1, ones_v, zflat,
              gsem0, gsem1, isem0, isem1, isem2, isem3):
    cid = lax.axis_index("c")
    sid = lax.axis_index("s")
    wid = sid * NC + cid
    gbufs = (rows0, rows1)
    gsems = (gsem0, gsem1)
    isems = (isem0, isem1, isem2, isem3)

    zv = jnp.zeros((L,), jnp.float32)
    ov = jnp.ones((L,), jnp.float32)

    @pl.loop(0, K)
    def _(r):
      for j in range(D // L):
        rows0[r, pl.ds(j * L, L)] = zv

    for j in range(K // L):
      ones_v[pl.ds(j * L, L)] = ov
    for j in range(rpt // L):
      zflat[pl.ds(j * L, L)] = zv

    for j in range(rpt // K):
      pltpu.sync_copy(rows0, acc.at[pl.ds(sid * rpt + j * K, K)])
    pltpu.sync_copy(zflat, dacc.at[pl.ds(sid * rpt, rpt)])
    plsc.subcore_barrier()

    ebase = wid * (nchunk * K)

    def start_idx(c, s):
      sl = pl.ds(ebase + c * K, K)
      pltpu.async_copy(src_hbm.at[sl], sring.at[s], isems[s])
      pltpu.async_copy(dst_hbm.at[sl], dring.at[s], isems[s])

    def start_gather(s, b):
      pltpu.make_async_copy(src_hbm.at[pl.ds(0, K)], sring.at[s], isems[s]).wait()
      pltpu.make_async_copy(dst_hbm.at[pl.ds(0, K)], dring.at[s], isems[s]).wait()
      pltpu.async_copy(x_hbm.at[sring.at[s]], gbufs[b], gsems[b])

    def accumulate(s, b):
      pltpu.make_async_copy(x_hbm.at[sring.at[s]], gbufs[b], gsems[b]).wait()
      pltpu.sync_copy(gbufs[b], acc.at[dring.at[s]], add=True)
      pltpu.sync_copy(ones_v, dacc.at[dring.at[s]], add=True)

    for c in range(4):
      start_idx(c, c)
    for c in range(2):
      start_gather(c, c)

    main = nchunk - 5
    assert main % 4 == 0

    @pl.loop(0, main, step=4)
    def _(i):
      for b in range(4):
        accumulate(b, b % 2)
        start_idx(i + b + 4, b)
        start_gather((b + 2) % 4, b % 2)

    accumulate(0, 0)
    start_idx(main + 4, 0)
    start_gather(2, 0)
    accumulate(1, 1)
    start_gather(3, 1)
    accumulate(2, 0)
    start_gather(0, 0)
    accumulate(3, 1)
    accumulate(0, 0)

    plsc.subcore_barrier()

    sl = pl.ds(sid * rpt, rpt)
    pltpu.sync_copy(acc.at[sl], s_out.at[cid, sl])
    pltpu.sync_copy(dacc.at[sl], deg_out.at[pl.ds(cid * NP + sid * rpt, rpt)])

  return seg_sum(x, src1d, dst1d)


def _dense(x, sp, degp, W, b2, R):
  N, D = x.shape
  grid = N // R

  def body(x_ref, sp_ref, degp_ref, w_ref, b_ref, o_ref):
    xb = x_ref[...]
    s = sp_ref[0] + sp_ref[1]
    deg = degp_ref[0, 0] + degp_ref[1, 0]
    w1 = w_ref[0:D]
    w2 = w_ref[D:2 * D]
    t = jnp.dot(s, w1, preferred_element_type=jnp.float32)
    u = jnp.dot(xb, w2, preferred_element_type=jnp.float32) + b_ref[0]
    z = t + deg * u
    sig = 1.0 / (1.0 + jnp.exp(-z))
    softplus = jnp.maximum(xb, 0.0) + jnp.log1p(jnp.exp(-jnp.abs(xb)))
    o_ref[...] = sig + softplus

  return pl.pallas_call(
      body,
      grid=(grid,),
      in_specs=[
          pl.BlockSpec((R, D), lambda i: (i, 0)),
          pl.BlockSpec((NC, R, D), lambda i: (0, i, 0)),
          pl.BlockSpec((NC, 1, R, 1), lambda i: (0, i, 0, 0)),
          pl.BlockSpec((2 * D, D), lambda i: (0, 0)),
          pl.BlockSpec((1, D), lambda i: (0, 0)),
      ],
      out_specs=pl.BlockSpec((R, D), lambda i: (i, 0)),
      out_shape=jax.ShapeDtypeStruct((N, D), jnp.float32),
  )(x, sp, degp, W, b2)


def kernel(node_feat, edge_index, W, b):
  N, D = node_feat.shape
  E = edge_index.shape[1]
  epw = E // NW
  nchunk = epw // K
  NP = -(-N // (NS * K)) * (NS * K)
  R = 1000

  sp, degp = _sc_segment_sum(node_feat, edge_index[0], edge_index[1], NP)
  degp4 = degp.reshape(NC, NP)[:, :N].reshape(NC, N // R, R, 1)
  return _dense(node_feat, sp, degp4, W, b.reshape(1, D), R)


# speedup vs baseline: 18.9699x; 1.1653x over previous
import functools

import jax
import jax.numpy as jnp
from jax import lax
from jax.experimental import pallas as pl
from jax.experimental.pallas import tpu as pltpu
from jax.experimental.pallas import tpu_sc as plsc

NC = 2
NS = 16
L = 16
NW = NC * NS

K = 80


def _sc_segment_sum(x, src1d, dst1d, NP):
  N, D = x.shape
  nchunk = src1d.shape[0] // (NW * K)
  rpt = NP // NS

  mesh = plsc.VectorSubcoreMesh(core_axis_name="c", subcore_axis_name="s")

  @functools.partial(
      pl.kernel,
      out_type=[
          jax.ShapeDtypeStruct((NC, NP, D), jnp.float32),
          jax.ShapeDtypeStruct((NC * NP,), jnp.float32),
      ],
      mesh=mesh,
      scratch_types=[
          pltpu.VMEM_SHARED((NP, D), jnp.float32),
          pltpu.VMEM_SHARED((NP,), jnp.float32),
          pltpu.VMEM((6, K), jnp.int32),
          pltpu.VMEM((6, K), jnp.int32),
          pltpu.VMEM((K, D), jnp.float32),
          pltpu.VMEM((K, D), jnp.float32),
          pltpu.VMEM((K, D), jnp.float32),
          pltpu.VMEM((K,), jnp.float32),
          pltpu.VMEM((rpt,), jnp.float32),
      ] + [pltpu.SemaphoreType.DMA] * 12,
  )
  def seg_sum(x_hbm, src_hbm, dst_hbm, s_out, deg_out, acc, dacc,
              sring, dring, rows0, rows1, rows2, ones_v, zflat, *sems):
    cid = lax.axis_index("c")
    sid = lax.axis_index("s")
    wid = sid * NC + cid
    rbufs = (rows0, rows1, rows2)
    gsems = sems[0:3]
    isems = sems[3:9]
    ssems = sems[9:12]

    zv = jnp.zeros((L,), jnp.float32)
    ov = jnp.ones((L,), jnp.float32)

    @pl.loop(0, K)
    def _(r):
      for j in range(D // L):
        rows0[r, pl.ds(j * L, L)] = zv

    for j in range(K // L):
      ones_v[pl.ds(j * L, L)] = ov
    for j in range(rpt // L):
      zflat[pl.ds(j * L, L)] = zv

    for j in range(rpt // K):
      pltpu.sync_copy(rows0, acc.at[pl.ds(sid * rpt + j * K, K)])
    pltpu.sync_copy(zflat, dacc.at[pl.ds(sid * rpt, rpt)])
    plsc.subcore_barrier()

    ebase = wid * (nchunk * K)

    def start_idx(c, s):
      sl = pl.ds(ebase + c * K, K)
      pltpu.async_copy(src_hbm.at[sl], sring.at[s], isems[s])
      pltpu.async_copy(dst_hbm.at[sl], dring.at[s], isems[s])

    def wait_idx(s):
      pltpu.make_async_copy(src_hbm.at[pl.ds(0, K)], sring.at[s], isems[s]).wait()
      pltpu.make_async_copy(dst_hbm.at[pl.ds(0, K)], dring.at[s], isems[s]).wait()

    def start_gather(s, b):
      pltpu.async_copy(x_hbm.at[sring.at[s]], rbufs[b], gsems[b])

    def wait_gather(s, b):
      pltpu.make_async_copy(x_hbm.at[sring.at[s]], rbufs[b], gsems[b]).wait()

    def start_scatter(s, b):
      pltpu.async_copy(rbufs[b], acc.at[dring.at[s]], ssems[b], add=True)
      pltpu.async_copy(ones_v, dacc.at[dring.at[s]], ssems[b], add=True)

    def wait_scatter(s, b):
      pltpu.make_async_copy(rbufs[b], acc.at[dring.at[s]], ssems[b]).wait()
      pltpu.make_async_copy(ones_v, dacc.at[dring.at[s]], ssems[b]).wait()

    def process(c, s, b, g2, i4, wprev):
      s2, b2, sp = (s + 2) % 6, (b + 2) % 3, (s + 5) % 6
      if g2:
        wait_idx(s2)
      if wprev:
        wait_scatter(sp, b2)
      if g2:
        start_gather(s2, b2)
      wait_gather(s, b)
      start_scatter(s, b)
      if i4:
        start_idx(c + 4, (s + 4) % 6)

    for c in range(4):
      start_idx(c, c)
    wait_idx(0)
    start_gather(0, 0)
    wait_idx(1)
    start_gather(1, 1)
    process(0, 0, 0, g2=True, i4=True, wprev=False)
    process(1, 1, 1, g2=True, i4=True, wprev=True)

    main = nchunk - 11
    assert main % 6 == 0

    @pl.loop(0, main, step=6)
    def _(i):
      for k in range(6):
        process(i + 2 + k, (2 + k) % 6, (2 + k) % 3, True, True, True)

    for c in range(main + 2, nchunk):
      process(c, c % 6, c % 3, g2=(c + 2 < nchunk), i4=(c + 4 < nchunk),
              wprev=True)
    wait_scatter((nchunk - 1) % 6, (nchunk - 1) % 3)

    plsc.subcore_barrier()

    sl = pl.ds(sid * rpt, rpt)
    pltpu.sync_copy(acc.at[sl], s_out.at[cid, sl])
    pltpu.sync_copy(dacc.at[sl], deg_out.at[pl.ds(cid * NP + sid * rpt, rpt)])

  return seg_sum(x, src1d, dst1d)


def _dense(x, sp, degp, W, b2, R):
  N, D = x.shape
  grid = N // R

  def body(x_ref, sp_ref, degp_ref, w_ref, b_ref, o_ref):
    xb = x_ref[...]
    s = sp_ref[0] + sp_ref[1]
    deg = degp_ref[0, 0] + degp_ref[1, 0]
    w1 = w_ref[0:D]
    w2 = w_ref[D:2 * D]
    t = jnp.dot(s, w1, preferred_element_type=jnp.float32)
    u = jnp.dot(xb, w2, preferred_element_type=jnp.float32) + b_ref[0]
    z = t + deg * u
    sig = 1.0 / (1.0 + jnp.exp(-z))
    softplus = jnp.maximum(xb, 0.0) + jnp.log1p(jnp.exp(-jnp.abs(xb)))
    o_ref[...] = sig + softplus

  return pl.pallas_call(
      body,
      grid=(grid,),
      in_specs=[
          pl.BlockSpec((R, D), lambda i: (i, 0)),
          pl.BlockSpec((NC, R, D), lambda i: (0, i, 0)),
          pl.BlockSpec((NC, 1, R, 1), lambda i: (0, i, 0, 0)),
          pl.BlockSpec((2 * D, D), lambda i: (0, 0)),
          pl.BlockSpec((1, D), lambda i: (0, 0)),
      ],
      out_specs=pl.BlockSpec((R, D), lambda i: (i, 0)),
      out_shape=jax.ShapeDtypeStruct((N, D), jnp.float32),
  )(x, sp, degp, W, b2)


def kernel(node_feat, edge_index, W, b):
  N, D = node_feat.shape
  E = edge_index.shape[1]
  epw = E // NW
  nchunk = epw // K
  NP = -(-N // (NS * K)) * (NS * K)
  R = 1000

  sp, degp = _sc_segment_sum(node_feat, edge_index[0], edge_index[1], NP)
  degp4 = degp.reshape(NC, NP)[:, :N].reshape(NC, N // R, R, 1)
  return _dense(node_feat, sp, degp4, W, b.reshape(1, D), R)
